# baseline (speedup 1.0000x reference)
import functools
import math

import jax
import jax.numpy as jnp
from jax.experimental import pallas as pl
from jax.experimental.pallas import tpu as pltpu
from jax.experimental.pallas import tpu_sc as plsc

B, T, N = 16, 24, 256
D = 128
M = 2048
K = 8
H = 4
HD = D // H
T_OUT = 25
KW = 12
TAU_TIME = 2.0

M_BLK = 128
O_BLK = 4096


def _gelu(x):
    return 0.5 * x * (1.0 + jax.lax.erf(x * (1.0 / math.sqrt(2.0))))


def _enc_kernel(x_ref, cw_ref, cb_ref, w_ref, b_ref, out_ref, h_scr):
    @pl.when(pl.program_id(0) == 0)
    def _():
        x = x_ref[...]
        acc = jnp.zeros((B, T_OUT, N), jnp.float32)
        for j in range(KW):
            acc = acc + x[:, j:j + T_OUT, :] * cw_ref[j]
        acc = _gelu(acc + cb_ref[...])
        h_scr[...] = jnp.transpose(acc, (1, 0, 2)).reshape(T_OUT * B, N)

    p = jax.lax.dot_general(h_scr[...], w_ref[...], (((1,), (1,)), ((), ())),
                            preferred_element_type=jnp.float32)
    p = _gelu(p + b_ref[...])
    acc = jnp.zeros((B, O_BLK), jnp.float32)
    for t in range(T_OUT):
        acc = acc + p[t * B:(t + 1) * B, :]
    out_ref[...] = acc * (1.0 / T_OUT)


def _simtop_kernel(qp_ref, mem_ref, msn_ref, myr_ref, sq_ref, yq_ref,
                   lnw_ref, lnb_ref, q_out, sim_out, qn_scr):
    m = pl.program_id(0)

    @pl.when(m == 0)
    def _():
        qp = qp_ref[...]
        mu = jnp.mean(qp, axis=-1, keepdims=True)
        var = jnp.mean((qp - mu) ** 2, axis=-1, keepdims=True)
        qln = (qp - mu) / jnp.sqrt(var + 1e-5) * lnw_ref[...] + lnb_ref[...]
        q_out[...] = qln
        qf = qln.reshape(B, N * D)
        qsq = jnp.sum(qf * qf, axis=1, keepdims=True)
        qn_scr[...] = qf / jnp.maximum(jnp.sqrt(qsq), 1e-12)

    mb = mem_ref[...].reshape(M_BLK, N * D)
    nsq = jnp.sum(mb * mb, axis=1, keepdims=True)
    mbn = mb / jnp.maximum(jnp.sqrt(nsq), 1e-12)
    sim = jax.lax.dot_general(mbn, qn_scr[...], (((1,), (1,)), ((), ())),
                              preferred_element_type=jnp.float32)
    mask = msn_ref[...] == sq_ref[...]
    sim = jnp.where(mask, sim, -10000.0)
    delta = jnp.abs(myr_ref[...] - yq_ref[...])
    div = 1.0 - jnp.exp(-delta / TAU_TIME)
    sim_out[...] = jnp.transpose(sim * (0.5 + 0.5 * div))


def _sc_topk_kernel(sim_hbm, out_hbm, row_v, out_v):
    cc = jax.lax.axis_index("c")
    ss = jax.lax.axis_index("s")
    b = ss * 2 + cc

    @pl.when(b < B)
    def _():
        pltpu.sync_copy(sim_hbm.at[b], row_v)
        lane = jax.lax.iota(jnp.int32, 16)
        dnums = jax.lax.GatherDimensionNumbers(
            offset_dims=(), collapsed_slice_dims=(0,), start_index_map=(0,))

        def shuf(v, sh):
            return jax.lax.gather(
                v, (lane ^ sh)[:, None], dimension_numbers=dnums,
                slice_sizes=(1,),
                mode=jax.lax.GatherScatterMode.PROMISE_IN_BOUNDS)

        def allreduce(v, op):
            for sh in (8, 4, 2, 1):
                v = op(v, shuf(v, sh))
            return v

        neg_inf = jnp.full((16,), -jnp.inf, jnp.float32)

        def body(ci, carry):
            vals, idxs = carry[:K], carry[K:]
            v = row_v[pl.ds(ci * 16, 16)]
            vm = ci * 16 + lane
            nvals, nidxs = [], []
            for i in range(K):
                cmp = v > vals[i]
                nvals.append(jnp.where(cmp, v, vals[i]))
                nidxs.append(jnp.where(cmp, vm, idxs[i]))
                v = jnp.where(cmp, vals[i], v)
                vm = jnp.where(cmp, idxs[i], vm)
            return (*nvals, *nidxs)

        carry = jax.lax.fori_loop(
            0, M // 16, body,
            (neg_inf,) * K + (jnp.full((16,), M, jnp.int32),) * K)
        vals, idxs = carry[:K], carry[K:]

        idx_acc = jnp.zeros((16,), jnp.int32)
        chosen = ()
        for r in range(K):
            mv, mi = neg_inf, jnp.full((16,), M, jnp.int32)
            for i in range(K):
                ok = jnp.full((16,), True)
                for cv in chosen:
                    ok = jnp.logical_and(ok, idxs[i] != cv)
                vi = jnp.where(ok, vals[i], neg_inf)
                cmp = vi > mv
                mi = jnp.where(cmp, idxs[i], mi)
                mv = jnp.where(cmp, vi, mv)
            mx_vec = allreduce(mv, jnp.maximum)
            cand = jnp.where(mv == mx_vec, mi, M)
            i_vec = allreduce(cand, jnp.minimum)
            idx_acc = jnp.where(lane == r, i_vec, idx_acc)
            chosen = chosen + (i_vec,)
        out_v[...] = idx_acc
        pltpu.sync_copy(out_v, out_hbm.at[b])


def _attn_kernel(idx_ref, q_ref, *refs):
    mem_refs = refs[:K]
    wq, wk, wv, bq, bk, bv, ow, ob, pw, pb, out_ref = refs[K:]
    gi = jax.lax.broadcasted_iota(jnp.int32, (D, H), 0)
    gj = jax.lax.broadcasted_iota(jnp.int32, (D, H), 1)
    g = (gi // HD == gj).astype(jnp.float32)
    gt = (jax.lax.broadcasted_iota(jnp.int32, (H, D), 1) // HD ==
          jax.lax.broadcasted_iota(jnp.int32, (H, D), 0)).astype(jnp.float32)

    qp = jnp.dot(q_ref[0], wq[...],
                 preferred_element_type=jnp.float32) + bq[...]
    r_all = jnp.concatenate([m[0] for m in mem_refs], axis=0)
    kp = jnp.dot(r_all, wk[...],
                 preferred_element_type=jnp.float32) + bk[...]
    vp = jnp.dot(r_all, wv[...],
                 preferred_element_type=jnp.float32) + bv[...]
    qp_t = jnp.concatenate([qp] * K, axis=0)
    prod = qp_t * kp * (1.0 / math.sqrt(HD))
    logits = jnp.dot(prod, g,
                     preferred_element_type=jnp.float32)
    logits = logits.reshape(K, N, H)
    mx = jnp.max(logits, axis=0, keepdims=True)
    e = jnp.exp(logits - mx)
    att = e / jnp.sum(e, axis=0, keepdims=True)
    att_exp = jnp.dot(att.reshape(K * N, H), gt,
                      preferred_element_type=jnp.float32)
    o = jnp.sum((att_exp * vp).reshape(K, N, D), axis=0)
    attn = jnp.dot(o, ow[...], preferred_element_type=jnp.float32) + ob[...]
    out_ref[0] = jnp.dot(attn, pw[...],
                         preferred_element_type=jnp.float32) + pb[...]


def kernel(x_scalar, season_q, year_q, dw_w, dw_b, pw_w, pw_b, ln_w, ln_b,
           in_proj_w, in_proj_b, out_proj_w, out_proj_b, proj_w, proj_b,
           memory_bank, memory_seasons, memory_years):
    f32 = jnp.float32
    x_scalar = x_scalar.astype(f32)
    season_q = season_q.astype(jnp.int32)
    year_q = year_q.astype(f32)
    memory_seasons = memory_seasons.astype(jnp.int32)
    memory_years = memory_years.astype(f32)

    x_pad = jnp.pad(x_scalar, ((0, 0), (6, 6), (0, 0)))
    w_t = jnp.transpose(dw_w[:, 0, :]).reshape(KW, 1, N)
    n_o = (N * D) // O_BLK
    q_pre = pl.pallas_call(
        _enc_kernel,
        grid=(n_o,),
        in_specs=[
            pl.BlockSpec((B, T + KW, N), lambda o: (0, 0, 0)),
            pl.BlockSpec((KW, 1, N), lambda o: (0, 0, 0)),
            pl.BlockSpec((1, 1, N), lambda o: (0, 0, 0)),
            pl.BlockSpec((O_BLK, N), lambda o: (o, 0)),
            pl.BlockSpec((1, O_BLK), lambda o: (0, o)),
        ],
        out_specs=pl.BlockSpec((B, O_BLK), lambda o: (0, o)),
        out_shape=jax.ShapeDtypeStruct((B, N * D), f32),
        scratch_shapes=[pltpu.VMEM((T_OUT * B, N), f32)],
    )(x_pad, w_t, dw_b.reshape(1, 1, N), pw_w, pw_b.reshape(1, N * D))

    n_m = M // M_BLK
    q3, sim_bt = pl.pallas_call(
        _simtop_kernel,
        grid=(n_m,),
        in_specs=[
            pl.BlockSpec((B, N, D), lambda m: (0, 0, 0)),
            pl.BlockSpec((M_BLK, N, D), lambda m: (m, 0, 0)),
            pl.BlockSpec((M_BLK, 1), lambda m: (m, 0)),
            pl.BlockSpec((M_BLK, 1), lambda m: (m, 0)),
            pl.BlockSpec((1, B), lambda m: (0, 0)),
            pl.BlockSpec((1, B), lambda m: (0, 0)),
            pl.BlockSpec((1, 1, D), lambda m: (0, 0, 0)),
            pl.BlockSpec((1, 1, D), lambda m: (0, 0, 0)),
        ],
        out_specs=[
            pl.BlockSpec((B, N, D), lambda m: (0, 0, 0)),
            pl.BlockSpec((B, M_BLK), lambda m: (0, m)),
        ],
        out_shape=[
            jax.ShapeDtypeStruct((B, N, D), f32),
            jax.ShapeDtypeStruct((B, M), f32),
        ],
        scratch_shapes=[pltpu.VMEM((B, N * D), f32)],
    )(q_pre.reshape(B, N, D), memory_bank, memory_seasons.reshape(M, 1),
      memory_years.reshape(M, 1), season_q.reshape(1, B),
      year_q.reshape(1, B), ln_w.reshape(1, 1, D), ln_b.reshape(1, 1, D))

    topk_idx = pl.kernel(
        _sc_topk_kernel,
        out_type=jax.ShapeDtypeStruct((B, 16), jnp.int32),
        mesh=plsc.VectorSubcoreMesh(core_axis_name="c", subcore_axis_name="s"),
        scratch_types=[pltpu.VMEM((M,), f32),
                       pltpu.VMEM((16,), jnp.int32)],
    )(sim_bt)

    wq_t = jnp.transpose(in_proj_w[:D])
    wk_t = jnp.transpose(in_proj_w[D:2 * D])
    wv_t = jnp.transpose(in_proj_w[2 * D:])
    bq = in_proj_b[:D].reshape(1, D)
    bk = in_proj_b[D:2 * D].reshape(1, D)
    bv = in_proj_b[2 * D:].reshape(1, D)
    ow_t = jnp.transpose(out_proj_w)
    pw_t = jnp.transpose(proj_w)

    mem_specs = [
        pl.BlockSpec((1, N, D), functools.partial(
            lambda b, idx, kk: (idx[b, kk], 0, 0), kk=k))
        for k in range(K)
    ]
    out = pl.pallas_call(
        _attn_kernel,
        grid_spec=pltpu.PrefetchScalarGridSpec(
            num_scalar_prefetch=1,
            grid=(B,),
            in_specs=[pl.BlockSpec((1, N, D), lambda b, idx: (b, 0, 0))]
            + mem_specs
            + [
                pl.BlockSpec((D, D), lambda b, idx: (0, 0)),
                pl.BlockSpec((D, D), lambda b, idx: (0, 0)),
                pl.BlockSpec((D, D), lambda b, idx: (0, 0)),
                pl.BlockSpec((1, D), lambda b, idx: (0, 0)),
                pl.BlockSpec((1, D), lambda b, idx: (0, 0)),
                pl.BlockSpec((1, D), lambda b, idx: (0, 0)),
                pl.BlockSpec((D, D), lambda b, idx: (0, 0)),
                pl.BlockSpec((1, D), lambda b, idx: (0, 0)),
                pl.BlockSpec((D, D), lambda b, idx: (0, 0)),
                pl.BlockSpec((1, D), lambda b, idx: (0, 0)),
            ],
            out_specs=pl.BlockSpec((1, N, D), lambda b, idx: (b, 0, 0)),
        ),
        out_shape=jax.ShapeDtypeStruct((B, N, D), f32),
    )(topk_idx, q3, *([memory_bank] * K), wq_t, wk_t, wv_t, bq, bk, bv,
      ow_t, out_proj_b.reshape(1, D), pw_t, proj_b.reshape(1, D))

    return (out, q3)


# speedup vs baseline: 1.0254x; 1.0254x over previous
import functools
import math

import jax
import jax.numpy as jnp
from jax.experimental import pallas as pl
from jax.experimental.pallas import tpu as pltpu
from jax.experimental.pallas import tpu_sc as plsc

B, T, N = 16, 24, 256
D = 128
M = 2048
K = 8
H = 4
HD = D // H
T_OUT = 25
KW = 12
TAU_TIME = 2.0

M_BLK = 128
O_BLK = 4096


def _gelu(x):
    return 0.5 * x * (1.0 + jax.lax.erf(x * (1.0 / math.sqrt(2.0))))


def _enc_kernel(x_ref, cw_ref, cb_ref, w_ref, b_ref, out_ref, h_scr):
    @pl.when(pl.program_id(0) == 0)
    def _():
        x = x_ref[...]
        acc = jnp.zeros((B, T_OUT, N), jnp.float32)
        for j in range(KW):
            acc = acc + x[:, j:j + T_OUT, :] * cw_ref[j]
        acc = _gelu(acc + cb_ref[...])
        h_scr[...] = jnp.transpose(acc, (1, 0, 2)).reshape(T_OUT * B, N)

    p = jax.lax.dot_general(h_scr[...], w_ref[...], (((1,), (1,)), ((), ())),
                            preferred_element_type=jnp.float32)
    p = _gelu(p + b_ref[...])
    acc = jnp.zeros((B, O_BLK), jnp.float32)
    for t in range(T_OUT):
        acc = acc + p[t * B:(t + 1) * B, :]
    out_ref[...] = acc * (1.0 / T_OUT)


def _simtop_kernel(qp_ref, mem_ref, msn_ref, myr_ref, sq_ref, yq_ref,
                   lnw_ref, lnb_ref, q_out, sim_out, qn_scr):
    m = pl.program_id(0)

    @pl.when(m == 0)
    def _():
        qp = qp_ref[...]
        mu = jnp.mean(qp, axis=-1, keepdims=True)
        var = jnp.mean((qp - mu) ** 2, axis=-1, keepdims=True)
        qln = (qp - mu) / jnp.sqrt(var + 1e-5) * lnw_ref[...] + lnb_ref[...]
        q_out[...] = qln
        qf = qln.reshape(B, N * D)
        qsq = jnp.sum(qf * qf, axis=1, keepdims=True)
        qn_scr[...] = qf / jnp.maximum(jnp.sqrt(qsq), 1e-12)

    mb = mem_ref[...].reshape(M_BLK, N * D)
    nsq = jnp.sum(mb * mb, axis=1, keepdims=True)
    mbn = mb / jnp.maximum(jnp.sqrt(nsq), 1e-12)
    sim = jax.lax.dot_general(mbn, qn_scr[...], (((1,), (1,)), ((), ())),
                              preferred_element_type=jnp.float32)
    mask = msn_ref[...] == sq_ref[...]
    sim = jnp.where(mask, sim, -10000.0)
    delta = jnp.abs(myr_ref[...] - yq_ref[...])
    div = 1.0 - jnp.exp(-delta / TAU_TIME)
    sim_out[...] = jnp.transpose(sim * (0.5 + 0.5 * div))


def _sc_topk_kernel(sim_hbm, out_hbm, row_v, out_v):
    cc = jax.lax.axis_index("c")
    ss = jax.lax.axis_index("s")
    b = ss * 2 + cc

    @pl.when(b < B)
    def _():
        pltpu.sync_copy(sim_hbm.at[b], row_v)
        lane = jax.lax.iota(jnp.int32, 16)
        dnums = jax.lax.GatherDimensionNumbers(
            offset_dims=(), collapsed_slice_dims=(0,), start_index_map=(0,))

        def shuf(v, sh):
            return jax.lax.gather(
                v, (lane ^ sh)[:, None], dimension_numbers=dnums,
                slice_sizes=(1,),
                mode=jax.lax.GatherScatterMode.PROMISE_IN_BOUNDS)

        def allreduce(v, op):
            for sh in (8, 4, 2, 1):
                v = op(v, shuf(v, sh))
            return v

        neg_inf = jnp.full((16,), -jnp.inf, jnp.float32)

        def body(ci, carry):
            vals, idxs = carry[:K], carry[K:]
            v = row_v[pl.ds(ci * 16, 16)]
            vm = ci * 16 + lane
            nvals, nidxs = [], []
            for i in range(K):
                cmp = v > vals[i]
                nvals.append(jnp.where(cmp, v, vals[i]))
                nidxs.append(jnp.where(cmp, vm, idxs[i]))
                v = jnp.where(cmp, vals[i], v)
                vm = jnp.where(cmp, idxs[i], vm)
            return (*nvals, *nidxs)

        carry = jax.lax.fori_loop(
            0, M // 16, body,
            (neg_inf,) * K + (jnp.full((16,), M, jnp.int32),) * K)
        vals, idxs = carry[:K], carry[K:]

        idx_acc = jnp.zeros((16,), jnp.int32)
        chosen = ()
        for r in range(K):
            mv, mi = neg_inf, jnp.full((16,), M, jnp.int32)
            for i in range(K):
                ok = jnp.full((16,), True)
                for cv in chosen:
                    ok = jnp.logical_and(ok, idxs[i] != cv)
                vi = jnp.where(ok, vals[i], neg_inf)
                cmp = vi > mv
                mi = jnp.where(cmp, idxs[i], mi)
                mv = jnp.where(cmp, vi, mv)
            mx_vec = allreduce(mv, jnp.maximum)
            cand = jnp.where(mv == mx_vec, mi, M)
            i_vec = allreduce(cand, jnp.minimum)
            idx_acc = jnp.where(lane == r, i_vec, idx_acc)
            chosen = chosen + (i_vec,)
        out_v[...] = idx_acc
        pltpu.sync_copy(out_v, out_hbm.at[b])


BB = 2


def _attn_kernel(idx_ref, q_ref, *refs):
    mem_refs = refs[:BB * K]
    wq, wk, wv, bq, bk, bv, ow, ob, pw, pb, out_ref = refs[BB * K:]
    gi = jax.lax.broadcasted_iota(jnp.int32, (D, H), 0)
    gj = jax.lax.broadcasted_iota(jnp.int32, (D, H), 1)
    g = (gi // HD == gj).astype(jnp.float32)
    gt = (jax.lax.broadcasted_iota(jnp.int32, (H, D), 1) // HD ==
          jax.lax.broadcasted_iota(jnp.int32, (H, D), 0)).astype(jnp.float32)

    R = BB * K * N
    qp = jnp.dot(q_ref[...].reshape(BB * N, D), wq[...],
                 preferred_element_type=jnp.float32) + bq[...]
    r_all = jnp.concatenate([m[0] for m in mem_refs], axis=0)
    kp = jnp.dot(r_all, wk[...],
                 preferred_element_type=jnp.float32) + bk[...]
    vp = jnp.dot(r_all, wv[...],
                 preferred_element_type=jnp.float32) + bv[...]
    qp_t = jnp.concatenate(
        [qp[bb * N:(bb + 1) * N] for bb in range(BB) for _ in range(K)],
        axis=0)
    prod = qp_t * kp * (1.0 / math.sqrt(HD))
    logits = jnp.dot(prod, g,
                     preferred_element_type=jnp.float32)
    logits = logits.reshape(BB, K, N, H)
    mx = jnp.max(logits, axis=1, keepdims=True)
    e = jnp.exp(logits - mx)
    att = e / jnp.sum(e, axis=1, keepdims=True)
    att_exp = jnp.dot(att.reshape(R, H), gt,
                      preferred_element_type=jnp.float32)
    o = jnp.sum((att_exp * vp).reshape(BB, K, N, D), axis=1)
    attn = jnp.dot(o.reshape(BB * N, D), ow[...],
                   preferred_element_type=jnp.float32) + ob[...]
    fin = jnp.dot(attn, pw[...],
                  preferred_element_type=jnp.float32) + pb[...]
    out_ref[...] = fin.reshape(BB, N, D)


def kernel(x_scalar, season_q, year_q, dw_w, dw_b, pw_w, pw_b, ln_w, ln_b,
           in_proj_w, in_proj_b, out_proj_w, out_proj_b, proj_w, proj_b,
           memory_bank, memory_seasons, memory_years):
    f32 = jnp.float32
    x_scalar = x_scalar.astype(f32)
    season_q = season_q.astype(jnp.int32)
    year_q = year_q.astype(f32)
    memory_seasons = memory_seasons.astype(jnp.int32)
    memory_years = memory_years.astype(f32)

    x_pad = jnp.pad(x_scalar, ((0, 0), (6, 6), (0, 0)))
    w_t = jnp.transpose(dw_w[:, 0, :]).reshape(KW, 1, N)
    n_o = (N * D) // O_BLK
    q_pre = pl.pallas_call(
        _enc_kernel,
        grid=(n_o,),
        in_specs=[
            pl.BlockSpec((B, T + KW, N), lambda o: (0, 0, 0)),
            pl.BlockSpec((KW, 1, N), lambda o: (0, 0, 0)),
            pl.BlockSpec((1, 1, N), lambda o: (0, 0, 0)),
            pl.BlockSpec((O_BLK, N), lambda o: (o, 0)),
            pl.BlockSpec((1, O_BLK), lambda o: (0, o)),
        ],
        out_specs=pl.BlockSpec((B, O_BLK), lambda o: (0, o)),
        out_shape=jax.ShapeDtypeStruct((B, N * D), f32),
        scratch_shapes=[pltpu.VMEM((T_OUT * B, N), f32)],
    )(x_pad, w_t, dw_b.reshape(1, 1, N), pw_w, pw_b.reshape(1, N * D))

    n_m = M // M_BLK
    q3, sim_bt = pl.pallas_call(
        _simtop_kernel,
        grid=(n_m,),
        in_specs=[
            pl.BlockSpec((B, N, D), lambda m: (0, 0, 0)),
            pl.BlockSpec((M_BLK, N, D), lambda m: (m, 0, 0)),
            pl.BlockSpec((M_BLK, 1), lambda m: (m, 0)),
            pl.BlockSpec((M_BLK, 1), lambda m: (m, 0)),
            pl.BlockSpec((1, B), lambda m: (0, 0)),
            pl.BlockSpec((1, B), lambda m: (0, 0)),
            pl.BlockSpec((1, 1, D), lambda m: (0, 0, 0)),
            pl.BlockSpec((1, 1, D), lambda m: (0, 0, 0)),
        ],
        out_specs=[
            pl.BlockSpec((B, N, D), lambda m: (0, 0, 0)),
            pl.BlockSpec((B, M_BLK), lambda m: (0, m)),
        ],
        out_shape=[
            jax.ShapeDtypeStruct((B, N, D), f32),
            jax.ShapeDtypeStruct((B, M), f32),
        ],
        scratch_shapes=[pltpu.VMEM((B, N * D), f32)],
    )(q_pre.reshape(B, N, D), memory_bank, memory_seasons.reshape(M, 1),
      memory_years.reshape(M, 1), season_q.reshape(1, B),
      year_q.reshape(1, B), ln_w.reshape(1, 1, D), ln_b.reshape(1, 1, D))

    topk_idx = pl.kernel(
        _sc_topk_kernel,
        out_type=jax.ShapeDtypeStruct((B, 16), jnp.int32),
        mesh=plsc.VectorSubcoreMesh(core_axis_name="c", subcore_axis_name="s"),
        scratch_types=[pltpu.VMEM((M,), f32),
                       pltpu.VMEM((16,), jnp.int32)],
    )(sim_bt)

    wq_t = jnp.transpose(in_proj_w[:D])
    wk_t = jnp.transpose(in_proj_w[D:2 * D])
    wv_t = jnp.transpose(in_proj_w[2 * D:])
    bq = in_proj_b[:D].reshape(1, D)
    bk = in_proj_b[D:2 * D].reshape(1, D)
    bv = in_proj_b[2 * D:].reshape(1, D)
    ow_t = jnp.transpose(out_proj_w)
    pw_t = jnp.transpose(proj_w)

    mem_specs = [
        pl.BlockSpec((1, N, D), functools.partial(
            lambda g, idx, bb, kk: (idx[g * BB + bb, kk], 0, 0),
            bb=j // K, kk=j % K))
        for j in range(BB * K)
    ]
    out = pl.pallas_call(
        _attn_kernel,
        grid_spec=pltpu.PrefetchScalarGridSpec(
            num_scalar_prefetch=1,
            grid=(B // BB,),
            in_specs=[pl.BlockSpec((BB, N, D), lambda g, idx: (g, 0, 0))]
            + mem_specs
            + [
                pl.BlockSpec((D, D), lambda b, idx: (0, 0)),
                pl.BlockSpec((D, D), lambda b, idx: (0, 0)),
                pl.BlockSpec((D, D), lambda b, idx: (0, 0)),
                pl.BlockSpec((1, D), lambda b, idx: (0, 0)),
                pl.BlockSpec((1, D), lambda b, idx: (0, 0)),
                pl.BlockSpec((1, D), lambda b, idx: (0, 0)),
                pl.BlockSpec((D, D), lambda b, idx: (0, 0)),
                pl.BlockSpec((1, D), lambda b, idx: (0, 0)),
                pl.BlockSpec((D, D), lambda b, idx: (0, 0)),
                pl.BlockSpec((1, D), lambda b, idx: (0, 0)),
            ],
            out_specs=pl.BlockSpec((BB, N, D), lambda g, idx: (g, 0, 0)),
        ),
        out_shape=jax.ShapeDtypeStruct((B, N, D), f32),
    )(topk_idx, q3, *([memory_bank] * (BB * K)), wq_t, wk_t, wv_t, bq, bk,
      bv, ow_t, out_proj_b.reshape(1, D), pw_t, proj_b.reshape(1, D))

    return (out, q3)


# speedup vs baseline: 1.0356x; 1.0099x over previous
import functools
import math

import jax
import jax.numpy as jnp
from jax.experimental import pallas as pl
from jax.experimental.pallas import tpu as pltpu
from jax.experimental.pallas import tpu_sc as plsc

B, T, N = 16, 24, 256
D = 128
M = 2048
K = 8
H = 4
HD = D // H
T_OUT = 25
KW = 12
TAU_TIME = 2.0

M_BLK = 128
O_BLK = 4096


def _gelu(x):
    return 0.5 * x * (1.0 + jax.lax.erf(x * (1.0 / math.sqrt(2.0))))


def _enc_kernel(x_ref, cw_ref, cb_ref, w_ref, b_ref, out_ref, h_scr):
    @pl.when(pl.program_id(0) == 0)
    def _():
        x = x_ref[...]
        acc = jnp.zeros((B, T_OUT, N), jnp.float32)
        for j in range(KW):
            acc = acc + x[:, j:j + T_OUT, :] * cw_ref[j]
        acc = _gelu(acc + cb_ref[...])
        h_scr[...] = jnp.transpose(acc, (1, 0, 2)).reshape(T_OUT * B, N)

    p = jax.lax.dot_general(h_scr[...], w_ref[...], (((1,), (1,)), ((), ())),
                            preferred_element_type=jnp.float32)
    p = _gelu(p + b_ref[...])
    acc = jnp.zeros((B, O_BLK), jnp.float32)
    for t in range(T_OUT):
        acc = acc + p[t * B:(t + 1) * B, :]
    out_ref[...] = acc * (1.0 / T_OUT)


def _simtop_kernel(qp_ref, mem_ref, msn_ref, myr_ref, sq_ref, yq_ref,
                   lnw_ref, lnb_ref, q_out, sim_out, qn_scr):
    m = pl.program_id(0)

    @pl.when(m == 0)
    def _():
        qp = qp_ref[...]
        mu = jnp.mean(qp, axis=-1, keepdims=True)
        var = jnp.mean((qp - mu) ** 2, axis=-1, keepdims=True)
        qln = (qp - mu) / jnp.sqrt(var + 1e-5) * lnw_ref[...] + lnb_ref[...]
        q_out[...] = qln
        qf = qln.reshape(B, N * D)
        qsq = jnp.sum(qf * qf, axis=1, keepdims=True)
        qn_scr[...] = qf / jnp.maximum(jnp.sqrt(qsq), 1e-12)

    mb = mem_ref[...].reshape(M_BLK, N * D)
    nsq = jnp.sum(mb * mb, axis=1, keepdims=True)
    mbn = mb / jnp.maximum(jnp.sqrt(nsq), 1e-12)
    sim = jax.lax.dot_general(mbn, qn_scr[...], (((1,), (1,)), ((), ())),
                              preferred_element_type=jnp.float32)
    mask = msn_ref[...] == sq_ref[...]
    sim = jnp.where(mask, sim, -10000.0)
    delta = jnp.abs(myr_ref[...] - yq_ref[...])
    div = 1.0 - jnp.exp(-delta / TAU_TIME)
    sim_out[...] = jnp.transpose(sim * (0.5 + 0.5 * div))


def _sc_topk_kernel(sim_hbm, out_hbm, row_v, out_v):
    cc = jax.lax.axis_index("c")
    ss = jax.lax.axis_index("s")
    b = ss * 2 + cc

    @pl.when(b < B)
    def _():
        pltpu.sync_copy(sim_hbm.at[b], row_v)
        lane = jax.lax.iota(jnp.int32, 16)
        dnums = jax.lax.GatherDimensionNumbers(
            offset_dims=(), collapsed_slice_dims=(0,), start_index_map=(0,))

        def shuf(v, sh):
            return jax.lax.gather(
                v, (lane ^ sh)[:, None], dimension_numbers=dnums,
                slice_sizes=(1,),
                mode=jax.lax.GatherScatterMode.PROMISE_IN_BOUNDS)

        def allreduce(v, op):
            for sh in (8, 4, 2, 1):
                v = op(v, shuf(v, sh))
            return v

        neg_inf = jnp.full((16,), -jnp.inf, jnp.float32)

        def body(ci, carry):
            vals, idxs = carry[:K], carry[K:]
            v = row_v[pl.ds(ci * 16, 16)]
            vm = ci * 16 + lane
            nvals, nidxs = [], []
            for i in range(K):
                cmp = v > vals[i]
                nvals.append(jnp.where(cmp, v, vals[i]))
                nidxs.append(jnp.where(cmp, vm, idxs[i]))
                v = jnp.where(cmp, vals[i], v)
                vm = jnp.where(cmp, idxs[i], vm)
            return (*nvals, *nidxs)

        carry = jax.lax.fori_loop(
            0, M // 16, body,
            (neg_inf,) * K + (jnp.full((16,), M, jnp.int32),) * K)
        vals, idxs = carry[:K], carry[K:]

        idx_acc = jnp.zeros((16,), jnp.int32)
        chosen = ()
        for r in range(K):
            mv, mi = neg_inf, jnp.full((16,), M, jnp.int32)
            for i in range(K):
                ok = jnp.full((16,), True)
                for cv in chosen:
                    ok = jnp.logical_and(ok, idxs[i] != cv)
                vi = jnp.where(ok, vals[i], neg_inf)
                cmp = vi > mv
                mi = jnp.where(cmp, idxs[i], mi)
                mv = jnp.where(cmp, vi, mv)
            mx_vec = allreduce(mv, jnp.maximum)
            cand = jnp.where(mv == mx_vec, mi, M)
            i_vec = allreduce(cand, jnp.minimum)
            idx_acc = jnp.where(lane == r, i_vec, idx_acc)
            chosen = chosen + (i_vec,)
        out_v[...] = idx_acc
        pltpu.sync_copy(out_v, out_hbm.at[b])


BB = 4


def _attn_kernel(idx_ref, q_ref, *refs):
    mem_refs = refs[:BB * K]
    wq, wk, wv, bq, bk, bv, ow, ob, pw, pb, out_ref = refs[BB * K:]
    gi = jax.lax.broadcasted_iota(jnp.int32, (D, H), 0)
    gj = jax.lax.broadcasted_iota(jnp.int32, (D, H), 1)
    g = (gi // HD == gj).astype(jnp.float32)
    gt = (jax.lax.broadcasted_iota(jnp.int32, (H, D), 1) // HD ==
          jax.lax.broadcasted_iota(jnp.int32, (H, D), 0)).astype(jnp.float32)

    R = BB * K * N
    qp = jnp.dot(q_ref[...].reshape(BB * N, D), wq[...],
                 preferred_element_type=jnp.float32) + bq[...]
    r_all = jnp.concatenate([m[0] for m in mem_refs], axis=0)
    kp = jnp.dot(r_all, wk[...],
                 preferred_element_type=jnp.float32) + bk[...]
    vp = jnp.dot(r_all, wv[...],
                 preferred_element_type=jnp.float32) + bv[...]
    qp_t = jnp.concatenate(
        [qp[bb * N:(bb + 1) * N] for bb in range(BB) for _ in range(K)],
        axis=0)
    prod = qp_t * kp * (1.0 / math.sqrt(HD))
    logits = jnp.dot(prod, g,
                     preferred_element_type=jnp.float32)
    logits = logits.reshape(BB, K, N, H)
    mx = jnp.max(logits, axis=1, keepdims=True)
    e = jnp.exp(logits - mx)
    att = e / jnp.sum(e, axis=1, keepdims=True)
    att_exp = jnp.dot(att.reshape(R, H), gt,
                      preferred_element_type=jnp.float32)
    o = jnp.sum((att_exp * vp).reshape(BB, K, N, D), axis=1)
    attn = jnp.dot(o.reshape(BB * N, D), ow[...],
                   preferred_element_type=jnp.float32) + ob[...]
    fin = jnp.dot(attn, pw[...],
                  preferred_element_type=jnp.float32) + pb[...]
    out_ref[...] = fin.reshape(BB, N, D)


def kernel(x_scalar, season_q, year_q, dw_w, dw_b, pw_w, pw_b, ln_w, ln_b,
           in_proj_w, in_proj_b, out_proj_w, out_proj_b, proj_w, proj_b,
           memory_bank, memory_seasons, memory_years):
    f32 = jnp.float32
    x_scalar = x_scalar.astype(f32)
    season_q = season_q.astype(jnp.int32)
    year_q = year_q.astype(f32)
    memory_seasons = memory_seasons.astype(jnp.int32)
    memory_years = memory_years.astype(f32)

    x_pad = jnp.pad(x_scalar, ((0, 0), (6, 6), (0, 0)))
    w_t = jnp.transpose(dw_w[:, 0, :]).reshape(KW, 1, N)
    n_o = (N * D) // O_BLK
    q_pre = pl.pallas_call(
        _enc_kernel,
        grid=(n_o,),
        in_specs=[
            pl.BlockSpec((B, T + KW, N), lambda o: (0, 0, 0)),
            pl.BlockSpec((KW, 1, N), lambda o: (0, 0, 0)),
            pl.BlockSpec((1, 1, N), lambda o: (0, 0, 0)),
            pl.BlockSpec((O_BLK, N), lambda o: (o, 0)),
            pl.BlockSpec((1, O_BLK), lambda o: (0, o)),
        ],
        out_specs=pl.BlockSpec((B, O_BLK), lambda o: (0, o)),
        out_shape=jax.ShapeDtypeStruct((B, N * D), f32),
        scratch_shapes=[pltpu.VMEM((T_OUT * B, N), f32)],
    )(x_pad, w_t, dw_b.reshape(1, 1, N), pw_w, pw_b.reshape(1, N * D))

    n_m = M // M_BLK
    q3, sim_bt = pl.pallas_call(
        _simtop_kernel,
        grid=(n_m,),
        in_specs=[
            pl.BlockSpec((B, N, D), lambda m: (0, 0, 0)),
            pl.BlockSpec((M_BLK, N, D), lambda m: (m, 0, 0)),
            pl.BlockSpec((M_BLK, 1), lambda m: (m, 0)),
            pl.BlockSpec((M_BLK, 1), lambda m: (m, 0)),
            pl.BlockSpec((1, B), lambda m: (0, 0)),
            pl.BlockSpec((1, B), lambda m: (0, 0)),
            pl.BlockSpec((1, 1, D), lambda m: (0, 0, 0)),
            pl.BlockSpec((1, 1, D), lambda m: (0, 0, 0)),
        ],
        out_specs=[
            pl.BlockSpec((B, N, D), lambda m: (0, 0, 0)),
            pl.BlockSpec((B, M_BLK), lambda m: (0, m)),
        ],
        out_shape=[
            jax.ShapeDtypeStruct((B, N, D), f32),
            jax.ShapeDtypeStruct((B, M), f32),
        ],
        scratch_shapes=[pltpu.VMEM((B, N * D), f32)],
    )(q_pre.reshape(B, N, D), memory_bank, memory_seasons.reshape(M, 1),
      memory_years.reshape(M, 1), season_q.reshape(1, B),
      year_q.reshape(1, B), ln_w.reshape(1, 1, D), ln_b.reshape(1, 1, D))

    topk_idx = pl.kernel(
        _sc_topk_kernel,
        out_type=jax.ShapeDtypeStruct((B, 16), jnp.int32),
        mesh=plsc.VectorSubcoreMesh(core_axis_name="c", subcore_axis_name="s"),
        scratch_types=[pltpu.VMEM((M,), f32),
                       pltpu.VMEM((16,), jnp.int32)],
    )(sim_bt)

    wq_t = jnp.transpose(in_proj_w[:D])
    wk_t = jnp.transpose(in_proj_w[D:2 * D])
    wv_t = jnp.transpose(in_proj_w[2 * D:])
    bq = in_proj_b[:D].reshape(1, D)
    bk = in_proj_b[D:2 * D].reshape(1, D)
    bv = in_proj_b[2 * D:].reshape(1, D)
    ow_t = jnp.transpose(out_proj_w)
    pw_t = jnp.transpose(proj_w)

    mem_specs = [
        pl.BlockSpec((1, N, D), functools.partial(
            lambda g, idx, bb, kk: (idx[g * BB + bb, kk], 0, 0),
            bb=j // K, kk=j % K))
        for j in range(BB * K)
    ]
    out = pl.pallas_call(
        _attn_kernel,
        grid_spec=pltpu.PrefetchScalarGridSpec(
            num_scalar_prefetch=1,
            grid=(B // BB,),
            in_specs=[pl.BlockSpec((BB, N, D), lambda g, idx: (g, 0, 0))]
            + mem_specs
            + [
                pl.BlockSpec((D, D), lambda b, idx: (0, 0)),
                pl.BlockSpec((D, D), lambda b, idx: (0, 0)),
                pl.BlockSpec((D, D), lambda b, idx: (0, 0)),
                pl.BlockSpec((1, D), lambda b, idx: (0, 0)),
                pl.BlockSpec((1, D), lambda b, idx: (0, 0)),
                pl.BlockSpec((1, D), lambda b, idx: (0, 0)),
                pl.BlockSpec((D, D), lambda b, idx: (0, 0)),
                pl.BlockSpec((1, D), lambda b, idx: (0, 0)),
                pl.BlockSpec((D, D), lambda b, idx: (0, 0)),
                pl.BlockSpec((1, D), lambda b, idx: (0, 0)),
            ],
            out_specs=pl.BlockSpec((BB, N, D), lambda g, idx: (g, 0, 0)),
        ),
        out_shape=jax.ShapeDtypeStruct((B, N, D), f32),
    )(topk_idx, q3, *([memory_bank] * (BB * K)), wq_t, wk_t, wv_t, bq, bk,
      bv, ow_t, out_proj_b.reshape(1, D), pw_t, proj_b.reshape(1, D))

    return (out, q3)
